# baseline (speedup 1.0000x reference)
import functools
import math

import jax
import jax.numpy as jnp
from jax import lax
from jax.experimental import pallas as pl
from jax.experimental.pallas import tpu as pltpu
from jax.experimental.pallas import tpu_sc as plsc

NC = 2
NS = 16
NW = NC * NS

CHUNK = 800
UNROLL = 2


def _scale_body(w_ref, scale_ref, o_ref):
    o_ref[...] = w_ref[...] * scale_ref[0]


def _prescale_table(weight, scale):
    V, D = weight.shape
    RB = 8000
    assert V % RB == 0
    return pl.pallas_call(
        _scale_body,
        out_shape=jax.ShapeDtypeStruct((V, D), jnp.float32),
        grid=(V // RB,),
        in_specs=[
            pl.BlockSpec((RB, D), lambda i: (i, 0)),
            pl.BlockSpec(memory_space=pltpu.SMEM),
        ],
        out_specs=pl.BlockSpec((RB, D), lambda i: (i, 0)),
    )(weight, jnp.full((1,), scale, dtype=jnp.float32))


def _make_gather(N, V, D):
    n_per_w = N // NW
    n_chunks = n_per_w // CHUNK
    n_iters = n_chunks // UNROLL
    assert n_per_w % CHUNK == 0 and n_chunks % UNROLL == 0
    rows_total = N // CHUNK

    mesh = plsc.VectorSubcoreMesh(
        core_axis_name="c", subcore_axis_name="s", num_cores=NC, num_subcores=NS
    )

    @functools.partial(
        pl.kernel,
        out_type=jax.ShapeDtypeStruct((rows_total, CHUNK, D), jnp.float32),
        mesh=mesh,
        compiler_params=pltpu.CompilerParams(use_tc_tiling_on_sc=False),
        scratch_types=[
            pltpu.VMEM((CHUNK,), jnp.int32),
            pltpu.VMEM((CHUNK,), jnp.int32),
            pltpu.VMEM((CHUNK, D), jnp.float32),
            pltpu.VMEM((CHUNK, D), jnp.float32),
            pltpu.SemaphoreType.DMA,
            pltpu.SemaphoreType.DMA,
            pltpu.SemaphoreType.DMA,
            pltpu.SemaphoreType.DMA,
        ],
    )
    def gather_k(tok_hbm, tab_hbm, out_hbm, idx0, idx1, rows0, rows1,
                 gsem0, gsem1, wsem0, wsem1):
        wid = lax.axis_index("s") * NC + lax.axis_index("c")
        base = wid * n_chunks

        def body(t, carry):
            c0 = base + UNROLL * t
            c1 = c0 + 1

            @pl.when(t > 0)
            def _():
                pltpu.make_async_copy(rows0, out_hbm.at[c0 - UNROLL], wsem0).wait()
                pltpu.make_async_copy(rows1, out_hbm.at[c1 - UNROLL], wsem1).wait()

            pltpu.sync_copy(tok_hbm.at[c0], idx0)
            g0 = pltpu.async_copy(tab_hbm.at[idx0], rows0, gsem0)
            pltpu.sync_copy(tok_hbm.at[c1], idx1)
            g1 = pltpu.async_copy(tab_hbm.at[idx1], rows1, gsem1)

            g0.wait()
            pltpu.async_copy(rows0, out_hbm.at[c0], wsem0)
            g1.wait()
            pltpu.async_copy(rows1, out_hbm.at[c1], wsem1)
            return carry

        lax.fori_loop(0, n_iters, body, 0)
        last0 = base + n_chunks - UNROLL
        pltpu.make_async_copy(rows0, out_hbm.at[last0], wsem0).wait()
        pltpu.make_async_copy(rows1, out_hbm.at[last0 + 1], wsem1).wait()

    return gather_k


def kernel(tokens, embedding_weight):
    B, S = tokens.shape
    V, D = embedding_weight.shape
    N = B * S

    scaled = _prescale_table(embedding_weight, math.sqrt(D))
    tok2d = tokens.astype(jnp.int32).reshape(N // CHUNK, CHUNK)
    out = _make_gather(N, V, D)(tok2d, scaled)
    return out.reshape(B, S, D)
